# baseline (speedup 1.0000x reference)
import functools

import jax
import jax.numpy as jnp
from jax.experimental import pallas as pl
from jax.experimental.pallas import tpu as pltpu

HDC_DECAY = 0.95
LN_EPS = 1e-5
_SQRT2 = 1.4142135623730951
_BB = 2


def _ln(x, g, b):
    mu = jnp.mean(x, axis=-1, keepdims=True)
    m2 = jnp.mean(x * x, axis=-1, keepdims=True)
    var = m2 - mu * mu
    return (x - mu) * jax.lax.rsqrt(var + LN_EPS) * g + b


def _gelu(x):
    return 0.5 * x * (1.0 + jax.lax.erf(x / _SQRT2))


def _dot(a, b):
    return jnp.dot(a.astype(jnp.bfloat16), b.astype(jnp.bfloat16),
                   preferred_element_type=jnp.float32)


def _mm(a3, w):
    bb, n, k = a3.shape
    return _dot(a3.reshape(bb * n, k), w).reshape(bb, n, -1)


def _agent_kernel(sig_ref, act_ref, ep_ref, cr_ref, prev_ref,
                  ln_pg_ref, ln_pb_ref, wp_ref, bp_ref,
                  ln_dg_ref, ln_db_ref, wd1_ref, bd1_ref, wd2_ref, bd2_ref,
                  wg_ref, bg_ref, wa_ref, ba_ref,
                  wth_ref, bth_ref, wfh_ref, bfh_ref,
                  wm_ref, bm_ref, pers_ref, char_ref, key_ref,
                  ln_og_ref, ln_ob_ref,
                  act_out_ref, st_out_ref, ne_hbm, eh_hbm, mg_out_ref,
                  wth_s, wfh_s, ne_buf, eh_buf, ne_sems, eh_sems):
    @pl.when(pl.program_id(0) == 0)
    def _stage_weights():
        d_dim, h_dim = wth_s.shape
        nch = 8
        ch = h_dim // nch

        def make_stager(w_hbm, w_bf16, col_chunks):
            def chunk_copy(stg, sems, c):
                if col_chunks:
                    src = w_hbm.at[:, c * ch:(c + 1) * ch]
                else:
                    src = w_hbm.at[c * ch:(c + 1) * ch, :]
                return pltpu.make_async_copy(src, stg.at[c % 2], sems.at[c % 2])

            def stage(stg, sems):
                chunk_copy(stg, sems, 0).start()
                for c in range(nch):
                    if c + 1 < nch:
                        chunk_copy(stg, sems, c + 1).start()
                    chunk_copy(stg, sems, c).wait()
                    cast = stg[c % 2].astype(jnp.bfloat16)
                    if col_chunks:
                        w_bf16[:, c * ch:(c + 1) * ch] = cast
                    else:
                        w_bf16[c * ch:(c + 1) * ch, :] = cast
            return stage

        pl.run_scoped(make_stager(wth_ref, wth_s, True),
                      pltpu.VMEM((2, d_dim, ch), jnp.float32),
                      pltpu.SemaphoreType.DMA((2,)))
        pl.run_scoped(make_stager(wfh_ref, wfh_s, False),
                      pltpu.VMEM((2, ch, d_dim), jnp.float32),
                      pltpu.SemaphoreType.DMA((2,)))

    sig = sig_ref[...]
    acts = act_ref[...]
    ep = ep_ref[...]
    cr = cr_ref[...]
    prev = prev_ref[...]
    pers = pers_ref[...]
    key = key_ref[...][None]

    full_p = pers + 0.3 * char_ref[...]
    mod = sig * (1.0 + 0.1 * full_p)[None]

    perc_in = jnp.concatenate([mod, acts], axis=-1)
    perc_n = _ln(perc_in, ln_pg_ref[...], ln_pb_ref[...])
    perception = _gelu(_mm(perc_n, wp_ref[...]) + bp_ref[...])

    hdc_query = jnp.tanh(_mm(perception, wth_s[...]) + bth_ref[...])
    pq = hdc_query * key

    epcr = ep + cr
    h_dim = epcr.shape[-1]
    sim = jnp.sum(epcr * pq, axis=-1, keepdims=True) / h_dim
    strength = jax.nn.sigmoid(sim)
    recall = (_mm(epcr, wfh_s[...]) + bfh_ref[...]) * strength

    dec_in = jnp.concatenate([perception, recall], axis=-1)
    dec_n = _ln(dec_in, ln_dg_ref[...], ln_db_ref[...])
    h = _gelu(_mm(dec_n, wd1_ref[...]) + bd1_ref[...])
    decision = (_mm(h, wd2_ref[...]) + bd2_ref[...]) * (1.0 + 0.1 * pers)[None]

    gate_in = jnp.concatenate([prev, decision], axis=-1)
    gate = jax.nn.sigmoid(_mm(gate_in, wg_ref[...]) + bg_ref[...])
    new_states = _ln(gate * decision + (1.0 - gate) * prev,
                     ln_og_ref[...], ln_ob_ref[...])
    act_out_ref[...] = _mm(new_states, wa_ref[...]) + ba_ref[...]
    st_out_ref[...] = new_states

    d_dim = sig.shape[-1]
    wms = wm_ref[:, :d_dim]
    wmh = wm_ref[:, d_dim:]
    mem_logit = (jnp.sum(new_states * wms, axis=-1, keepdims=True)
                 + jnp.sum(hdc_query * wmh, axis=-1, keepdims=True)
                 + bm_ref[0, 0])
    mem_gate = jax.nn.sigmoid(mem_logit)
    mg_out_ref[...] = mem_gate

    exp_hdc = jnp.tanh(_mm(new_states, wth_s[...]) + bth_ref[...])
    new_ep = HDC_DECAY * ep + mem_gate * (exp_hdc * key)

    i = pl.program_id(0)
    nsteps = pl.num_programs(0)
    bb = sig.shape[0]

    def out_copy(buf, hbm, sems, slot, step):
        return pltpu.make_async_copy(
            buf.at[slot], hbm.at[pl.ds(step * bb, bb)], sems.at[slot])

    for slot in range(2):
        @pl.when(jax.lax.rem(i, 2) == slot)
        def _write():
            @pl.when(i >= 2)
            def _drain_prev():
                out_copy(ne_buf, ne_hbm, ne_sems, slot, i - 2).wait()
                out_copy(eh_buf, eh_hbm, eh_sems, slot, i - 2).wait()
            ne_buf.at[slot][...] = new_ep
            eh_buf.at[slot][...] = exp_hdc
            out_copy(ne_buf, ne_hbm, ne_sems, slot, i).start()
            out_copy(eh_buf, eh_hbm, eh_sems, slot, i).start()

    @pl.when(i == nsteps - 1)
    def _drain_tail():
        for slot in range(2):
            step = i - jax.lax.rem(i + slot, 2)
            out_copy(ne_buf, ne_hbm, ne_sems, slot, step).wait()
            out_copy(eh_buf, eh_hbm, eh_sems, slot, step).wait()


@functools.partial(jax.jit, static_argnames=())
def kernel(incoming_signal, incoming_actions, episodic_mem, crystallized_mem,
           prev_states, step,
           ln_perc_g, ln_perc_b, W_perc, b_perc,
           ln_dec_g, ln_dec_b, W_dec1, b_dec1, W_dec2, b_dec2,
           W_gate, b_gate, W_act, b_act,
           W_to_hdc, b_to_hdc, W_from_hdc, b_from_hdc,
           W_mem, b_mem, personality, character, agent_key,
           ln_out_g, ln_out_b):
    del step
    B, N, D = incoming_signal.shape
    H = episodic_mem.shape[-1]

    row = lambda v: v.reshape(1, -1)
    wm = row(W_mem)
    bm = b_mem.reshape(1, 1)

    blk_d = lambda: pl.BlockSpec((_BB, N, D), lambda i: (i, 0, 0))
    blk_h = lambda: pl.BlockSpec((_BB, N, H), lambda i: (i, 0, 0))
    full = lambda a: pl.BlockSpec(a.shape, lambda i: (0,) * a.ndim)

    small_ins = (row(ln_perc_g), row(ln_perc_b), W_perc, row(b_perc),
                 row(ln_dec_g), row(ln_dec_b), W_dec1, row(b_dec1),
                 W_dec2, row(b_dec2), W_gate, row(b_gate), W_act, row(b_act),
                 W_to_hdc, row(b_to_hdc), W_from_hdc, row(b_from_hdc),
                 wm, bm, personality, character, agent_key,
                 row(ln_out_g), row(ln_out_b))
    spec_of = {id(W_to_hdc): pl.BlockSpec(memory_space=pl.ANY),
               id(W_from_hdc): pl.BlockSpec(memory_space=pl.ANY)}

    out = pl.pallas_call(
        _agent_kernel,
        grid=(B // _BB,),
        in_specs=[blk_d(), blk_d(), blk_h(), blk_h(), blk_d()]
                 + [spec_of.get(id(a), full(a)) for a in small_ins],
        scratch_shapes=[pltpu.VMEM((D, H), jnp.bfloat16),
                        pltpu.VMEM((H, D), jnp.bfloat16),
                        pltpu.VMEM((2, _BB, N, H), jnp.float32),
                        pltpu.VMEM((2, _BB, N, H), jnp.float32),
                        pltpu.SemaphoreType.DMA((2,)),
                        pltpu.SemaphoreType.DMA((2,))],
        out_specs=[blk_d(), blk_d(),
                   pl.BlockSpec(memory_space=pl.ANY),
                   pl.BlockSpec(memory_space=pl.ANY),
                   pl.BlockSpec((_BB, N, 1), lambda i: (i, 0, 0))],
        out_shape=[
            jax.ShapeDtypeStruct((B, N, D), jnp.float32),
            jax.ShapeDtypeStruct((B, N, D), jnp.float32),
            jax.ShapeDtypeStruct((B, N, H), jnp.float32),
            jax.ShapeDtypeStruct((B, N, H), jnp.float32),
            jax.ShapeDtypeStruct((B, N, 1), jnp.float32),
        ],
        compiler_params=pltpu.CompilerParams(
            dimension_semantics=("arbitrary",),
            vmem_limit_bytes=57 * 1024 * 1024,
        ),
        name="shared_conscious_agent",
    )(incoming_signal, incoming_actions, episodic_mem, crystallized_mem,
      prev_states, *small_ins)
    actions, new_states, new_episodics, exp_hdc, mem_gate = out
    return (actions, new_states, new_episodics, exp_hdc, mem_gate)


# speedup vs baseline: 1.0577x; 1.0577x over previous
import functools

import jax
import jax.numpy as jnp
from jax.experimental import pallas as pl
from jax.experimental.pallas import tpu as pltpu

HDC_DECAY = 0.95
LN_EPS = 1e-5
_SQRT2 = 1.4142135623730951
_BB = 2


def _ln(x, g, b):
    mu = jnp.mean(x, axis=-1, keepdims=True)
    m2 = jnp.mean(x * x, axis=-1, keepdims=True)
    var = m2 - mu * mu
    return (x - mu) * jax.lax.rsqrt(var + LN_EPS) * g + b


def _gelu(x):
    return 0.5 * x * (1.0 + jax.lax.erf(x / _SQRT2))


def _dot(a, b):
    return jnp.dot(a.astype(jnp.bfloat16), b.astype(jnp.bfloat16),
                   preferred_element_type=jnp.float32)


def _mm(a3, w):
    bb, n, k = a3.shape
    return _dot(a3.reshape(bb * n, k), w).reshape(bb, n, -1)


def _agent_kernel(sig_ref, act_ref, ep_ref, cr_ref, prev_ref,
                  ln_pg_ref, ln_pb_ref, wp_ref, bp_ref,
                  ln_dg_ref, ln_db_ref, wd1_ref, bd1_ref, wd2_ref, bd2_ref,
                  wg_ref, bg_ref, wa_ref, ba_ref,
                  wth_ref, bth_ref, wfh_ref, bfh_ref,
                  wm_ref, bm_ref, pers_ref, char_ref, key_ref,
                  ln_og_ref, ln_ob_ref,
                  act_out_ref, st_out_ref, ne_out_ref, eh_out_ref, mg_out_ref,
                  wth_s, wfh_s):
    @pl.when(pl.program_id(0) == 0)
    def _stage_weights():
        d_dim, h_dim = wth_s.shape
        nch = 4
        ch = h_dim // nch

        def make_stager(w_hbm, w_bf16, col_chunks):
            def chunk_copy(stg, sems, c):
                if col_chunks:
                    src = w_hbm.at[:, c * ch:(c + 1) * ch]
                else:
                    src = w_hbm.at[c * ch:(c + 1) * ch, :]
                return pltpu.make_async_copy(src, stg.at[c % 2], sems.at[c % 2])

            def stage(stg, sems):
                chunk_copy(stg, sems, 0).start()
                for c in range(nch):
                    if c + 1 < nch:
                        chunk_copy(stg, sems, c + 1).start()
                    chunk_copy(stg, sems, c).wait()
                    cast = stg[c % 2].astype(jnp.bfloat16)
                    if col_chunks:
                        w_bf16[:, c * ch:(c + 1) * ch] = cast
                    else:
                        w_bf16[c * ch:(c + 1) * ch, :] = cast
            return stage

        pl.run_scoped(make_stager(wth_ref, wth_s, True),
                      pltpu.VMEM((2, d_dim, ch), jnp.float32),
                      pltpu.SemaphoreType.DMA((2,)))
        pl.run_scoped(make_stager(wfh_ref, wfh_s, False),
                      pltpu.VMEM((2, ch, d_dim), jnp.float32),
                      pltpu.SemaphoreType.DMA((2,)))

    sig = sig_ref[...]
    acts = act_ref[...]
    ep = ep_ref[...]
    cr = cr_ref[...]
    prev = prev_ref[...]
    pers = pers_ref[...]
    key = key_ref[...][None]

    full_p = pers + 0.3 * char_ref[...]
    mod = sig * (1.0 + 0.1 * full_p)[None]

    perc_in = jnp.concatenate([mod, acts], axis=-1)
    perc_n = _ln(perc_in, ln_pg_ref[...], ln_pb_ref[...])
    perception = _gelu(_mm(perc_n, wp_ref[...]) + bp_ref[...])

    hdc_query = jnp.tanh(_mm(perception, wth_s[...]) + bth_ref[...])
    pq = hdc_query * key

    epcr = ep + cr
    h_dim = epcr.shape[-1]
    sim = jnp.sum(epcr * pq, axis=-1, keepdims=True) / h_dim
    strength = jax.nn.sigmoid(sim)
    recall = (_mm(epcr, wfh_s[...]) + bfh_ref[...]) * strength

    dec_in = jnp.concatenate([perception, recall], axis=-1)
    dec_n = _ln(dec_in, ln_dg_ref[...], ln_db_ref[...])
    h = _gelu(_mm(dec_n, wd1_ref[...]) + bd1_ref[...])
    decision = (_mm(h, wd2_ref[...]) + bd2_ref[...]) * (1.0 + 0.1 * pers)[None]

    gate_in = jnp.concatenate([prev, decision], axis=-1)
    gate = jax.nn.sigmoid(_mm(gate_in, wg_ref[...]) + bg_ref[...])
    new_states = _ln(gate * decision + (1.0 - gate) * prev,
                     ln_og_ref[...], ln_ob_ref[...])
    act_out_ref[...] = _mm(new_states, wa_ref[...]) + ba_ref[...]
    st_out_ref[...] = new_states

    d_dim = sig.shape[-1]
    wms = wm_ref[:, :d_dim]
    wmh = wm_ref[:, d_dim:]
    mem_logit = (jnp.sum(new_states * wms, axis=-1, keepdims=True)
                 + jnp.sum(hdc_query * wmh, axis=-1, keepdims=True)
                 + bm_ref[0, 0])
    mem_gate = jax.nn.sigmoid(mem_logit)
    mg_out_ref[...] = mem_gate

    exp_hdc = jnp.tanh(_mm(new_states, wth_s[...]) + bth_ref[...])
    eh_out_ref[...] = exp_hdc
    ne_out_ref[...] = HDC_DECAY * ep + mem_gate * (exp_hdc * key)


@functools.partial(jax.jit, static_argnames=())
def kernel(incoming_signal, incoming_actions, episodic_mem, crystallized_mem,
           prev_states, step,
           ln_perc_g, ln_perc_b, W_perc, b_perc,
           ln_dec_g, ln_dec_b, W_dec1, b_dec1, W_dec2, b_dec2,
           W_gate, b_gate, W_act, b_act,
           W_to_hdc, b_to_hdc, W_from_hdc, b_from_hdc,
           W_mem, b_mem, personality, character, agent_key,
           ln_out_g, ln_out_b):
    del step
    B, N, D = incoming_signal.shape
    H = episodic_mem.shape[-1]

    row = lambda v: v.reshape(1, -1)
    wm = row(W_mem)
    bm = b_mem.reshape(1, 1)

    blk_d = lambda: pl.BlockSpec((_BB, N, D), lambda i: (i, 0, 0))
    blk_h = lambda: pl.BlockSpec((_BB, N, H), lambda i: (i, 0, 0))
    full = lambda a: pl.BlockSpec(a.shape, lambda i: (0,) * a.ndim)

    small_ins = (row(ln_perc_g), row(ln_perc_b), W_perc, row(b_perc),
                 row(ln_dec_g), row(ln_dec_b), W_dec1, row(b_dec1),
                 W_dec2, row(b_dec2), W_gate, row(b_gate), W_act, row(b_act),
                 W_to_hdc, row(b_to_hdc), W_from_hdc, row(b_from_hdc),
                 wm, bm, personality, character, agent_key,
                 row(ln_out_g), row(ln_out_b))
    spec_of = {id(W_to_hdc): pl.BlockSpec(memory_space=pl.ANY),
               id(W_from_hdc): pl.BlockSpec(memory_space=pl.ANY)}

    out = pl.pallas_call(
        _agent_kernel,
        grid=(B // _BB,),
        in_specs=[blk_d(), blk_d(), blk_h(), blk_h(), blk_d()]
                 + [spec_of.get(id(a), full(a)) for a in small_ins],
        scratch_shapes=[pltpu.VMEM((D, H), jnp.bfloat16),
                        pltpu.VMEM((H, D), jnp.bfloat16)],
        out_specs=[blk_d(), blk_d(), blk_h(), blk_h(),
                   pl.BlockSpec((_BB, N, 1), lambda i: (i, 0, 0))],
        out_shape=[
            jax.ShapeDtypeStruct((B, N, D), jnp.float32),
            jax.ShapeDtypeStruct((B, N, D), jnp.float32),
            jax.ShapeDtypeStruct((B, N, H), jnp.float32),
            jax.ShapeDtypeStruct((B, N, H), jnp.float32),
            jax.ShapeDtypeStruct((B, N, 1), jnp.float32),
        ],
        compiler_params=pltpu.CompilerParams(
            dimension_semantics=("arbitrary",),
            vmem_limit_bytes=57 * 1024 * 1024,
        ),
        name="shared_conscious_agent",
    )(incoming_signal, incoming_actions, episodic_mem, crystallized_mem,
      prev_states, *small_ins)
    actions, new_states, new_episodics, exp_hdc, mem_gate = out
    return (actions, new_states, new_episodics, exp_hdc, mem_gate)
